# initial kernel scaffold (unmeasured)
import jax
import jax.numpy as jnp
from jax import lax
from jax.experimental import pallas as pl
from jax.experimental.pallas import tpu as pltpu

N_DEV = 8
ROWS = 8192
COLS = 1024
CHUNK = ROWS // N_DEV
NSLOTS = 4


def _allreduce_body(p_ref, out_ref, comm_ref,
                    rs_send_sems, rs_recv_sems, ag_send_sems, ag_recv_sems):
    my = lax.axis_index("i")
    left = lax.rem(my + N_DEV - 1, N_DEV)
    right = lax.rem(my + 1, N_DEV)

    barrier_sem = pltpu.get_barrier_semaphore()
    for nbr in (left, right):
        pl.semaphore_signal(
            barrier_sem, inc=1,
            device_id=(nbr,), device_id_type=pl.DeviceIdType.MESH,
        )
    pl.semaphore_wait(barrier_sem, 2)


    for s in range(N_DEV - 1):
        send_chunk = lax.rem(my + N_DEV - s, N_DEV)
        recv_chunk = lax.rem(my + N_DEV - s - 1, N_DEV)
        slot = s % NSLOTS
        rdma = pltpu.make_async_remote_copy(
            src_ref=out_ref.at[pl.ds(send_chunk * CHUNK, CHUNK), :],
            dst_ref=comm_ref.at[slot],
            send_sem=rs_send_sems.at[s],
            recv_sem=rs_recv_sems.at[s],
            device_id=(right,),
            device_id_type=pl.DeviceIdType.MESH,
        )
        rdma.start()
        rdma.wait()
        rows = pl.ds(recv_chunk * CHUNK, CHUNK)
        out_ref[rows, :] = out_ref[rows, :] + comm_ref[slot]

    for s in range(N_DEV - 1):
        send_chunk = lax.rem(my + 1 + N_DEV - s, N_DEV)
        rdma = pltpu.make_async_remote_copy(
            src_ref=out_ref.at[pl.ds(send_chunk * CHUNK, CHUNK), :],
            dst_ref=out_ref.at[pl.ds(send_chunk * CHUNK, CHUNK), :],
            send_sem=ag_send_sems.at[s],
            recv_sem=ag_recv_sems.at[s],
            device_id=(right,),
            device_id_type=pl.DeviceIdType.MESH,
        )
        rdma.start()
        rdma.wait()


def _ring_allreduce(partial):
    return pl.pallas_call(
        _allreduce_body,
        out_shape=jax.ShapeDtypeStruct((ROWS, COLS), jnp.float32),
        in_specs=[pl.BlockSpec(memory_space=pltpu.VMEM)],
        out_specs=pl.BlockSpec(memory_space=pltpu.VMEM),
        scratch_shapes=[
            pltpu.VMEM((NSLOTS, CHUNK, COLS), jnp.float32),
            pltpu.SemaphoreType.DMA((N_DEV - 1,)),
            pltpu.SemaphoreType.DMA((N_DEV - 1,)),
            pltpu.SemaphoreType.DMA((N_DEV - 1,)),
            pltpu.SemaphoreType.DMA((N_DEV - 1,)),
        ],
        input_output_aliases={0: 0},
        compiler_params=pltpu.CompilerParams(collective_id=0),
    )(partial)


def kernel(x, k, Wp):
    b, seq, c = x.shape
    taps = k.shape[0]

    pad = jnp.pad(x, ((0, 0), (taps - 1, 0), (0, 0)))
    out = pad[:, 0:seq, :] * k[0][None, None, :]
    for t in range(1, taps):
        out = out + pad[:, t:t + seq, :] * k[t][None, None, :]

    a = out * jax.nn.sigmoid(out)

    partial = lax.dot_general(
        a.reshape(b * seq, c).astype(jnp.bfloat16),
        Wp.astype(jnp.bfloat16),
        (((1,), (0,)), ((), ())),
        preferred_element_type=jnp.float32,
    )

    reduced = _ring_allreduce(partial)
    return reduced.reshape(b, seq, COLS)


# baseline (device time: 776149 ns/iter reference)
import jax
import jax.numpy as jnp
from jax import lax
from jax.experimental import pallas as pl
from jax.experimental.pallas import tpu as pltpu

N_DEV = 8
ROWS = 8192
COLS = 1024
CHUNK = ROWS // N_DEV
NSLOTS = 4


def _allreduce_body(p_ref, out_ref, comm_ref, copy_sem,
                    rs_send_sems, rs_recv_sems, ag_send_sems, ag_recv_sems):
    my = lax.axis_index("i")
    left = lax.rem(my + N_DEV - 1, N_DEV)
    right = lax.rem(my + 1, N_DEV)

    load = pltpu.make_async_copy(p_ref, out_ref, copy_sem)
    load.start()

    barrier_sem = pltpu.get_barrier_semaphore()
    for nbr in (left, right):
        pl.semaphore_signal(
            barrier_sem, inc=1,
            device_id=(nbr,), device_id_type=pl.DeviceIdType.MESH,
        )
    pl.semaphore_wait(barrier_sem, 2)
    load.wait()

    for s in range(N_DEV - 1):
        send_chunk = lax.rem(my + N_DEV - s, N_DEV)
        recv_chunk = lax.rem(my + N_DEV - s - 1, N_DEV)
        slot = s % NSLOTS
        rdma = pltpu.make_async_remote_copy(
            src_ref=out_ref.at[pl.ds(send_chunk * CHUNK, CHUNK), :],
            dst_ref=comm_ref.at[slot],
            send_sem=rs_send_sems.at[s],
            recv_sem=rs_recv_sems.at[s],
            device_id=(right,),
            device_id_type=pl.DeviceIdType.MESH,
        )
        rdma.start()
        rdma.wait()
        rows = pl.ds(recv_chunk * CHUNK, CHUNK)
        out_ref[rows, :] = out_ref[rows, :] + comm_ref[slot]

    for s in range(N_DEV - 1):
        send_chunk = lax.rem(my + 1 + N_DEV - s, N_DEV)
        rdma = pltpu.make_async_remote_copy(
            src_ref=out_ref.at[pl.ds(send_chunk * CHUNK, CHUNK), :],
            dst_ref=out_ref.at[pl.ds(send_chunk * CHUNK, CHUNK), :],
            send_sem=ag_send_sems.at[s],
            recv_sem=ag_recv_sems.at[s],
            device_id=(right,),
            device_id_type=pl.DeviceIdType.MESH,
        )
        rdma.start()
        rdma.wait()


def _ring_allreduce(partial):
    return pl.pallas_call(
        _allreduce_body,
        out_shape=jax.ShapeDtypeStruct((ROWS, COLS), jnp.float32),
        in_specs=[pl.BlockSpec(memory_space=pl.ANY)],
        out_specs=pl.BlockSpec(memory_space=pltpu.VMEM),
        scratch_shapes=[
            pltpu.VMEM((NSLOTS, CHUNK, COLS), jnp.float32),
            pltpu.SemaphoreType.DMA,
            pltpu.SemaphoreType.DMA((N_DEV - 1,)),
            pltpu.SemaphoreType.DMA((N_DEV - 1,)),
            pltpu.SemaphoreType.DMA((N_DEV - 1,)),
            pltpu.SemaphoreType.DMA((N_DEV - 1,)),
        ],
        compiler_params=pltpu.CompilerParams(
            collective_id=0, vmem_limit_bytes=60 * 1024 * 1024,
        ),
    )(partial)


def kernel(x, k, Wp):
    b, seq, c = x.shape
    taps = k.shape[0]

    pad = jnp.pad(x, ((0, 0), (taps - 1, 0), (0, 0)))
    out = pad[:, 0:seq, :] * k[0][None, None, :]
    for t in range(1, taps):
        out = out + pad[:, t:t + seq, :] * k[t][None, None, :]

    a = out * jax.nn.sigmoid(out)

    partial = lax.dot_general(
        a.reshape(b * seq, c).astype(jnp.bfloat16),
        Wp.astype(jnp.bfloat16),
        (((1,), (0,)), ((), ())),
        preferred_element_type=jnp.float32,
    )

    reduced = _ring_allreduce(partial)
    return reduced.reshape(b, seq, COLS)


# device time: 285616 ns/iter; 2.7175x vs baseline; 2.7175x over previous
import jax
import jax.numpy as jnp
from jax import lax
from jax.experimental import pallas as pl
from jax.experimental.pallas import tpu as pltpu

N_DEV = 8
ROWS = 8192
COLS = 1024
HALF = ROWS // 2
CHUNK = HALF // N_DEV
NSLOTS = 4
CDT = jnp.bfloat16


def _allreduce_body(p_ref, out_ref, comm_ref, copy_sem,
                    rs_send, rs_recv, ag_send, ag_recv):
    my = lax.axis_index("i")
    left = lax.rem(my + N_DEV - 1, N_DEV)
    right = lax.rem(my + 1, N_DEV)

    load = pltpu.make_async_copy(p_ref, out_ref, copy_sem)
    load.start()

    barrier_sem = pltpu.get_barrier_semaphore()
    for nbr in (left, right):
        pl.semaphore_signal(
            barrier_sem, inc=1,
            device_id=(nbr,), device_id_type=pl.DeviceIdType.MESH,
        )
    pl.semaphore_wait(barrier_sem, 2)
    load.wait()

    for s in range(N_DEV - 1):
        slot = s % NSLOTS
        sc_r = lax.rem(my + N_DEV - s, N_DEV)
        rc_r = lax.rem(my + N_DEV - s - 1, N_DEV)
        sc_l = lax.rem(my + s, N_DEV)
        rc_l = lax.rem(my + s + 1, N_DEV)
        rdma_r = pltpu.make_async_remote_copy(
            src_ref=out_ref.at[pl.ds(sc_r * CHUNK, CHUNK), :],
            dst_ref=comm_ref.at[0, slot],
            send_sem=rs_send.at[0, slot],
            recv_sem=rs_recv.at[0, slot],
            device_id=(right,),
            device_id_type=pl.DeviceIdType.MESH,
        )
        rdma_l = pltpu.make_async_remote_copy(
            src_ref=out_ref.at[pl.ds(HALF + sc_l * CHUNK, CHUNK), :],
            dst_ref=comm_ref.at[1, slot],
            send_sem=rs_send.at[1, slot],
            recv_sem=rs_recv.at[1, slot],
            device_id=(left,),
            device_id_type=pl.DeviceIdType.MESH,
        )
        rdma_r.start()
        rdma_l.start()
        rdma_r.wait()
        rows_r = pl.ds(rc_r * CHUNK, CHUNK)
        out_ref[rows_r, :] = out_ref[rows_r, :] + comm_ref[0, slot]
        rdma_l.wait()
        rows_l = pl.ds(HALF + rc_l * CHUNK, CHUNK)
        out_ref[rows_l, :] = out_ref[rows_l, :] + comm_ref[1, slot]

    for s in range(N_DEV - 1):
        slot = s % NSLOTS
        sc_r = lax.rem(my + 1 + N_DEV - s, N_DEV)
        sc_l = lax.rem(my + N_DEV - 1 + s, N_DEV)
        rdma_r = pltpu.make_async_remote_copy(
            src_ref=out_ref.at[pl.ds(sc_r * CHUNK, CHUNK), :],
            dst_ref=out_ref.at[pl.ds(sc_r * CHUNK, CHUNK), :],
            send_sem=ag_send.at[0, slot],
            recv_sem=ag_recv.at[0, slot],
            device_id=(right,),
            device_id_type=pl.DeviceIdType.MESH,
        )
        rdma_l = pltpu.make_async_remote_copy(
            src_ref=out_ref.at[pl.ds(HALF + sc_l * CHUNK, CHUNK), :],
            dst_ref=out_ref.at[pl.ds(HALF + sc_l * CHUNK, CHUNK), :],
            send_sem=ag_send.at[1, slot],
            recv_sem=ag_recv.at[1, slot],
            device_id=(left,),
            device_id_type=pl.DeviceIdType.MESH,
        )
        rdma_r.start()
        rdma_l.start()
        rdma_r.wait()
        rdma_l.wait()


def _ring_allreduce(partial):
    return pl.pallas_call(
        _allreduce_body,
        out_shape=jax.ShapeDtypeStruct((ROWS, COLS), CDT),
        in_specs=[pl.BlockSpec(memory_space=pl.ANY)],
        out_specs=pl.BlockSpec(memory_space=pltpu.VMEM),
        scratch_shapes=[
            pltpu.VMEM((2, NSLOTS, CHUNK, COLS), CDT),
            pltpu.SemaphoreType.DMA,
            pltpu.SemaphoreType.DMA((2, NSLOTS)),
            pltpu.SemaphoreType.DMA((2, NSLOTS)),
            pltpu.SemaphoreType.DMA((2, NSLOTS)),
            pltpu.SemaphoreType.DMA((2, NSLOTS)),
        ],
        compiler_params=pltpu.CompilerParams(
            collective_id=0, vmem_limit_bytes=60 * 1024 * 1024,
        ),
    )(partial)


def kernel(x, k, Wp):
    b, seq, c = x.shape
    taps = k.shape[0]

    pad = jnp.pad(x, ((0, 0), (taps - 1, 0), (0, 0)))
    out = pad[:, 0:seq, :] * k[0][None, None, :]
    for t in range(1, taps):
        out = out + pad[:, t:t + seq, :] * k[t][None, None, :]

    a = out * jax.nn.sigmoid(out)

    partial = lax.dot_general(
        a.reshape(b * seq, c).astype(jnp.bfloat16),
        Wp.astype(jnp.bfloat16),
        (((1,), (0,)), ((), ())),
        preferred_element_type=jnp.float32,
    ).astype(CDT)

    reduced = _ring_allreduce(partial)
    return reduced.reshape(b, seq, COLS)


# device time: 276614 ns/iter; 2.8059x vs baseline; 1.0325x over previous
import jax
import jax.numpy as jnp
from jax import lax
from jax.experimental import pallas as pl
from jax.experimental.pallas import tpu as pltpu

N_DEV = 8
ROWS = 8192
COLS = 1024
HALF = ROWS // 2
CHUNK = HALF // N_DEV
NSLOTS = 4
CDT = jnp.bfloat16


def _allreduce_body(p_ref, out_ref, comm_ref, copy_sem,
                    rs_send, rs_recv, ag_send, ag_recv):
    my = lax.axis_index("i")
    left = lax.rem(my + N_DEV - 1, N_DEV)
    right = lax.rem(my + 1, N_DEV)

    load = pltpu.make_async_copy(p_ref, out_ref, copy_sem)
    load.start()

    barrier_sem = pltpu.get_barrier_semaphore()
    for nbr in (left, right):
        pl.semaphore_signal(
            barrier_sem, inc=1,
            device_id=(nbr,), device_id_type=pl.DeviceIdType.MESH,
        )
    pl.semaphore_wait(barrier_sem, 2)
    load.wait()

    for s in range(N_DEV - 1):
        slot = s % NSLOTS
        sc_r = lax.rem(my + N_DEV - s, N_DEV)
        rc_r = lax.rem(my + N_DEV - s - 1, N_DEV)
        sc_l = lax.rem(my + s, N_DEV)
        rc_l = lax.rem(my + s + 1, N_DEV)
        rdma_r = pltpu.make_async_remote_copy(
            src_ref=out_ref.at[pl.ds(sc_r * CHUNK, CHUNK), :],
            dst_ref=comm_ref.at[0, slot],
            send_sem=rs_send.at[0, slot],
            recv_sem=rs_recv.at[0, slot],
            device_id=(right,),
            device_id_type=pl.DeviceIdType.MESH,
        )
        rdma_l = pltpu.make_async_remote_copy(
            src_ref=out_ref.at[pl.ds(HALF + sc_l * CHUNK, CHUNK), :],
            dst_ref=comm_ref.at[1, slot],
            send_sem=rs_send.at[1, slot],
            recv_sem=rs_recv.at[1, slot],
            device_id=(left,),
            device_id_type=pl.DeviceIdType.MESH,
        )
        rdma_r.start()
        rdma_l.start()
        rdma_r.wait()
        rows_r = pl.ds(rc_r * CHUNK, CHUNK)
        out_ref[rows_r, :] = out_ref[rows_r, :] + comm_ref[0, slot]
        rdma_l.wait()
        rows_l = pl.ds(HALF + rc_l * CHUNK, CHUNK)
        out_ref[rows_l, :] = out_ref[rows_l, :] + comm_ref[1, slot]

    for s in range(N_DEV - 1):
        slot = s % NSLOTS
        sc_r = lax.rem(my + 1 + N_DEV - s, N_DEV)
        sc_l = lax.rem(my + N_DEV - 1 + s, N_DEV)
        rdma_r = pltpu.make_async_remote_copy(
            src_ref=out_ref.at[pl.ds(sc_r * CHUNK, CHUNK), :],
            dst_ref=out_ref.at[pl.ds(sc_r * CHUNK, CHUNK), :],
            send_sem=ag_send.at[0, slot],
            recv_sem=ag_recv.at[0, slot],
            device_id=(right,),
            device_id_type=pl.DeviceIdType.MESH,
        )
        rdma_l = pltpu.make_async_remote_copy(
            src_ref=out_ref.at[pl.ds(HALF + sc_l * CHUNK, CHUNK), :],
            dst_ref=out_ref.at[pl.ds(HALF + sc_l * CHUNK, CHUNK), :],
            send_sem=ag_send.at[1, slot],
            recv_sem=ag_recv.at[1, slot],
            device_id=(left,),
            device_id_type=pl.DeviceIdType.MESH,
        )
        rdma_r.start()
        rdma_l.start()
        rdma_r.wait()
        rdma_l.wait()


def _ring_allreduce(partial):
    return pl.pallas_call(
        _allreduce_body,
        out_shape=jax.ShapeDtypeStruct((ROWS, COLS), CDT),
        in_specs=[pl.BlockSpec(memory_space=pl.ANY)],
        out_specs=pl.BlockSpec(memory_space=pltpu.VMEM),
        scratch_shapes=[
            pltpu.VMEM((2, NSLOTS, CHUNK, COLS), CDT),
            pltpu.SemaphoreType.DMA,
            pltpu.SemaphoreType.DMA((2, NSLOTS)),
            pltpu.SemaphoreType.DMA((2, NSLOTS)),
            pltpu.SemaphoreType.DMA((2, NSLOTS)),
            pltpu.SemaphoreType.DMA((2, NSLOTS)),
        ],
        compiler_params=pltpu.CompilerParams(
            collective_id=0, vmem_limit_bytes=60 * 1024 * 1024,
        ),
    )(partial)


def kernel(x, k, Wp):
    b, seq, c = x.shape
    taps = k.shape[0]

    xb = x.astype(jnp.bfloat16)
    kb = k.astype(jnp.bfloat16)
    pad = jnp.pad(xb, ((0, 0), (taps - 1, 0), (0, 0)))
    out = pad[:, 0:seq, :] * kb[0][None, None, :]
    for t in range(1, taps):
        out = out + pad[:, t:t + seq, :] * kb[t][None, None, :]

    a = out * jax.nn.sigmoid(out)

    partial = lax.dot_general(
        a.reshape(b * seq, c),
        Wp.astype(jnp.bfloat16),
        (((1,), (0,)), ((), ())),
        preferred_element_type=jnp.float32,
    ).astype(CDT)

    reduced = _ring_allreduce(partial)
    return reduced.reshape(b, seq, COLS)


# device time: 208150 ns/iter; 3.7288x vs baseline; 1.3289x over previous
import jax
import jax.numpy as jnp
from jax import lax
from jax.experimental import pallas as pl
from jax.experimental.pallas import tpu as pltpu

N_DEV = 8
ROWS = 8192
COLS = 1024
CDT = jnp.bfloat16

PARTS = (2816, 2688, 2688)
BASES = (0, 2816, 5504)
MASKS = ((1, 3, 4), (3, 4, 1), (4, 1, 3))
BUF_ROWS = max(PARTS) // 2


def _allreduce_body(p_ref, out_ref, buf_ref, copy_sem,
                    rs_send, rs_recv, ag_send, ag_recv):
    my = lax.axis_index("i")

    load = pltpu.make_async_copy(p_ref, out_ref, copy_sem)
    load.start()

    barrier_sem = pltpu.get_barrier_semaphore()
    for mask in (1, 3, 4):
        pl.semaphore_signal(
            barrier_sem, inc=1,
            device_id=(jnp.bitwise_xor(my, mask),),
            device_id_type=pl.DeviceIdType.MESH,
        )
    pl.semaphore_wait(barrier_sem, 3)
    load.wait()

    los = [jnp.int32(BASES[p]) for p in range(3)]
    lens = [PARTS[p] for p in range(3)]

    side_of = {
        1: jnp.bitwise_and(jnp.bitwise_xor(my, my >> 1), 1),
        3: jnp.bitwise_and(my >> 1, 1),
        4: jnp.bitwise_and(my >> 2, 1),
    }

    for s in range(3):
        rdmas = []
        for p in range(3):
            mask = MASKS[p][s]
            q = jnp.bitwise_xor(my, mask)
            side = side_of[mask] == 1
            h = lens[p] // 2
            send_start = los[p] + jnp.where(side, 0, h)
            rdma = pltpu.make_async_remote_copy(
                src_ref=out_ref.at[pl.ds(send_start, h), :],
                dst_ref=buf_ref.at[p, pl.ds(0, h), :],
                send_sem=rs_send.at[p, s],
                recv_sem=rs_recv.at[p, s],
                device_id=(q,),
                device_id_type=pl.DeviceIdType.MESH,
            )
            rdma.start()
            rdmas.append(rdma)
            los[p] = los[p] + jnp.where(side, h, 0)
            lens[p] = h
        for p in range(3):
            rdmas[p].wait()
            h = lens[p]
            rows = pl.ds(los[p], h)
            out_ref[rows, :] = out_ref[rows, :] + buf_ref[p, pl.ds(0, h), :]

    for s in (2, 1, 0):
        rdmas = []
        for p in range(3):
            mask = MASKS[p][s]
            q = jnp.bitwise_xor(my, mask)
            side = side_of[mask] == 1
            L = lens[p]
            rdma = pltpu.make_async_remote_copy(
                src_ref=out_ref.at[pl.ds(los[p], L), :],
                dst_ref=out_ref.at[pl.ds(los[p], L), :],
                send_sem=ag_send.at[p, s],
                recv_sem=ag_recv.at[p, s],
                device_id=(q,),
                device_id_type=pl.DeviceIdType.MESH,
            )
            rdma.start()
            rdmas.append(rdma)
            los[p] = los[p] - jnp.where(side, L, 0)
            lens[p] = 2 * L
        for p in range(3):
            rdmas[p].wait()


def _allreduce(partial):
    return pl.pallas_call(
        _allreduce_body,
        out_shape=jax.ShapeDtypeStruct((ROWS, COLS), CDT),
        in_specs=[pl.BlockSpec(memory_space=pl.ANY)],
        out_specs=pl.BlockSpec(memory_space=pltpu.VMEM),
        scratch_shapes=[
            pltpu.VMEM((3, BUF_ROWS, COLS), CDT),
            pltpu.SemaphoreType.DMA,
            pltpu.SemaphoreType.DMA((3, 3)),
            pltpu.SemaphoreType.DMA((3, 3)),
            pltpu.SemaphoreType.DMA((3, 3)),
            pltpu.SemaphoreType.DMA((3, 3)),
        ],
        compiler_params=pltpu.CompilerParams(
            collective_id=0, vmem_limit_bytes=60 * 1024 * 1024,
        ),
    )(partial)


def kernel(x, k, Wp):
    b, seq, c = x.shape
    taps = k.shape[0]

    xb = x.astype(jnp.bfloat16)
    kb = k.astype(jnp.bfloat16)
    pad = jnp.pad(xb, ((0, 0), (taps - 1, 0), (0, 0)))
    out = pad[:, 0:seq, :] * kb[0][None, None, :]
    for t in range(1, taps):
        out = out + pad[:, t:t + seq, :] * kb[t][None, None, :]

    a = out * jax.nn.sigmoid(out)

    partial = lax.dot_general(
        a.reshape(b * seq, c),
        Wp.astype(jnp.bfloat16),
        (((1,), (0,)), ((), ())),
        preferred_element_type=jnp.float32,
    ).astype(CDT)

    reduced = _allreduce(partial)
    return reduced.reshape(b, seq, COLS)


# device time: 192898 ns/iter; 4.0236x vs baseline; 1.0791x over previous
import jax
import jax.numpy as jnp
from jax import lax
from jax.experimental import pallas as pl
from jax.experimental.pallas import tpu as pltpu

N_DEV = 8
ROWS = 8192
SEQ = 2048
COLS = 1024
TAPS = 4
CDT = jnp.bfloat16
XPAD = 16

PARTS = (2816, 2688, 2688)
BASES = (0, 2816, 5504)
MASKS = ((1, 3, 4), (3, 4, 1), (4, 1, 3))
BUF_ROWS = max(PARTS) // 2


def _body(x_ref, k_ref, wp_ref, out_ref, xpad_ref, buf_ref, copy_sem,
          rs_send, rs_recv, ag_send, ag_recv):
    my = lax.axis_index("i")

    xpad_ref[0:XPAD, :] = jnp.zeros((XPAD, COLS), CDT)
    load = pltpu.make_async_copy(
        x_ref, xpad_ref.at[pl.ds(XPAD, ROWS), :], copy_sem)
    load.start()

    barrier_sem = pltpu.get_barrier_semaphore()
    for mask in (1, 3, 4):
        pl.semaphore_signal(
            barrier_sem, inc=1,
            device_id=(jnp.bitwise_xor(my, mask),),
            device_id_type=pl.DeviceIdType.MESH,
        )
    pl.semaphore_wait(barrier_sem, 3)
    load.wait()

    def compute_block(r0, h):
        i = lax.broadcasted_iota(jnp.int32, (h, 1), 0)
        s = lax.rem(r0 + i, SEQ)
        blk = xpad_ref[pl.ds(r0 + XPAD, h), :]
        prev = xpad_ref[pl.ds(r0 + XPAD - 8, h), :]
        acc = None
        for t in range(TAPS):
            off = TAPS - 1 - t
            if off:
                seg = jnp.where(
                    i < h - 8 + off,
                    pltpu.roll(prev, h - 8 + off, axis=0),
                    pltpu.roll(blk, off, axis=0),
                )
                seg = jnp.where(s >= off, seg, jnp.bfloat16(0.0))
            else:
                seg = blk
            term = seg * k_ref[t:t + 1, :]
            acc = term if acc is None else acc + term
        a = acc * jax.nn.sigmoid(acc)
        out = jnp.dot(a, wp_ref[:, :], preferred_element_type=jnp.float32)
        out_ref[pl.ds(r0, h), :] = out.astype(CDT)

    side_of = {
        1: jnp.bitwise_and(jnp.bitwise_xor(my, my >> 1), 1),
        3: jnp.bitwise_and(my >> 1, 1),
        4: jnp.bitwise_and(my >> 2, 1),
    }

    los = [None, None, None]
    lens = [PARTS[p] // 2 for p in range(3)]

    rdmas0 = []
    for p in range(3):
        mask = MASKS[p][0]
        q = jnp.bitwise_xor(my, mask)
        side = side_of[mask] == 1
        h = lens[p]
        send_start = BASES[p] + jnp.where(side, 0, h)
        compute_block(send_start, h)
        rdma = pltpu.make_async_remote_copy(
            src_ref=out_ref.at[pl.ds(send_start, h), :],
            dst_ref=buf_ref.at[p, pl.ds(0, h), :],
            send_sem=rs_send.at[p, 0],
            recv_sem=rs_recv.at[p, 0],
            device_id=(q,),
            device_id_type=pl.DeviceIdType.MESH,
        )
        rdma.start()
        rdmas0.append(rdma)
        los[p] = BASES[p] + jnp.where(side, h, 0)

    for p in range(3):
        compute_block(los[p], lens[p])
    for p in range(3):
        rdmas0[p].wait()
        h = lens[p]
        rows = pl.ds(los[p], h)
        out_ref[rows, :] = out_ref[rows, :] + buf_ref[p, pl.ds(0, h), :]

    for s in (1, 2):
        rdmas = []
        for p in range(3):
            mask = MASKS[p][s]
            q = jnp.bitwise_xor(my, mask)
            side = side_of[mask] == 1
            h = lens[p] // 2
            send_start = los[p] + jnp.where(side, 0, h)
            rdma = pltpu.make_async_remote_copy(
                src_ref=out_ref.at[pl.ds(send_start, h), :],
                dst_ref=buf_ref.at[p, pl.ds(0, h), :],
                send_sem=rs_send.at[p, s],
                recv_sem=rs_recv.at[p, s],
                device_id=(q,),
                device_id_type=pl.DeviceIdType.MESH,
            )
            rdma.start()
            rdmas.append(rdma)
            los[p] = los[p] + jnp.where(side, h, 0)
            lens[p] = h
        for p in range(3):
            rdmas[p].wait()
            h = lens[p]
            rows = pl.ds(los[p], h)
            out_ref[rows, :] = out_ref[rows, :] + buf_ref[p, pl.ds(0, h), :]

    for s in (2, 1, 0):
        rdmas = []
        for p in range(3):
            mask = MASKS[p][s]
            q = jnp.bitwise_xor(my, mask)
            side = side_of[mask] == 1
            L = lens[p]
            rdma = pltpu.make_async_remote_copy(
                src_ref=out_ref.at[pl.ds(los[p], L), :],
                dst_ref=out_ref.at[pl.ds(los[p], L), :],
                send_sem=ag_send.at[p, s],
                recv_sem=ag_recv.at[p, s],
                device_id=(q,),
                device_id_type=pl.DeviceIdType.MESH,
            )
            rdma.start()
            rdmas.append(rdma)
            los[p] = los[p] - jnp.where(side, L, 0)
            lens[p] = 2 * L
        for p in range(3):
            rdmas[p].wait()


def kernel(x, k, Wp):
    b, seq, c = x.shape
    xb = x.reshape(b * seq, c).astype(CDT)
    kb = k.astype(CDT)
    wpb = Wp.astype(CDT)

    reduced = pl.pallas_call(
        _body,
        out_shape=jax.ShapeDtypeStruct((ROWS, COLS), CDT),
        in_specs=[
            pl.BlockSpec(memory_space=pl.ANY),
            pl.BlockSpec(memory_space=pltpu.VMEM),
            pl.BlockSpec(memory_space=pltpu.VMEM),
        ],
        out_specs=pl.BlockSpec(memory_space=pltpu.VMEM),
        scratch_shapes=[
            pltpu.VMEM((ROWS + XPAD, COLS), CDT),
            pltpu.VMEM((3, BUF_ROWS, COLS), CDT),
            pltpu.SemaphoreType.DMA,
            pltpu.SemaphoreType.DMA((3, 3)),
            pltpu.SemaphoreType.DMA((3, 3)),
            pltpu.SemaphoreType.DMA((3, 3)),
            pltpu.SemaphoreType.DMA((3, 3)),
        ],
        compiler_params=pltpu.CompilerParams(
            collective_id=0, vmem_limit_bytes=62 * 1024 * 1024,
        ),
    )(xb, kb, wpb)
    return reduced.reshape(b, seq, COLS)


# device time: 171823 ns/iter; 4.5171x vs baseline; 1.1227x over previous
import jax
import jax.numpy as jnp
from jax import lax
from jax.experimental import pallas as pl
from jax.experimental.pallas import tpu as pltpu

N_DEV = 8
ROWS = 8192
SEQ = 2048
COLS = 1024
TAPS = 4
CDT = jnp.bfloat16

PARTS = (2816, 2688, 2688)
BASES = (0, 2816, 5504)
MASKS = ((1, 3, 4), (3, 4, 1), (4, 1, 3))
BUF_ROWS = max(PARTS) // 2
STAG_ROWS = BUF_ROWS + 16


def _body(x_ref, k_ref, wp_ref, out_ref, stag_ref, buf_ref, load_sems,
          rs_send, rs_recv, ag_send, ag_recv):
    my = lax.axis_index("i")

    barrier_sem = pltpu.get_barrier_semaphore()
    for mask in (1, 3, 4):
        pl.semaphore_signal(
            barrier_sem, inc=1,
            device_id=(jnp.bitwise_xor(my, mask),),
            device_id_type=pl.DeviceIdType.MESH,
        )

    side_of = {
        1: jnp.bitwise_and(jnp.bitwise_xor(my, my >> 1), 1),
        3: jnp.bitwise_and(my >> 1, 1),
        4: jnp.bitwise_and(my >> 2, 1),
    }

    lens = [PARTS[p] // 2 for p in range(3)]
    send_starts, keep_starts = [], []
    for p in range(3):
        side = side_of[MASKS[p][0]] == 1
        h = lens[p]
        send_starts.append(BASES[p] + jnp.where(side, 0, h))
        keep_starts.append(BASES[p] + jnp.where(side, h, 0))
    los = list(keep_starts)

    blocks = [(send_starts[p], lens[p]) for p in range(3)] + \
             [(keep_starts[p], lens[p]) for p in range(3)]

    def start_load(b, slot):
        r0, h = blocks[b]
        src0 = pl.multiple_of(jnp.maximum(r0 - 8, 0), 8)
        dst0 = pl.multiple_of(jnp.where(r0 == 0, 8, 0), 8)
        cp = pltpu.make_async_copy(
            x_ref.at[pl.ds(src0, h + 8), :],
            stag_ref.at[slot, pl.ds(dst0, h + 8), :],
            load_sems.at[slot],
        )
        cp.start()
        return cp

    def compute_block(b, slot):
        r0, h = blocks[b]
        i = lax.broadcasted_iota(jnp.int32, (h, 1), 0)
        s = lax.rem(r0 + i, SEQ)
        blk = stag_ref[slot, pl.ds(8, h), :].astype(CDT)
        prev = stag_ref[slot, pl.ds(0, h), :].astype(CDT)
        acc = None
        for t in range(TAPS):
            off = TAPS - 1 - t
            if off:
                seg = jnp.where(
                    i < h - 8 + off,
                    pltpu.roll(prev, h - 8 + off, axis=0),
                    pltpu.roll(blk, off, axis=0),
                )
                seg = jnp.where(s >= off, seg, jnp.bfloat16(0.0))
            else:
                seg = blk
            term = seg * k_ref[t:t + 1, :]
            acc = term if acc is None else acc + term
        a = acc * jax.nn.sigmoid(acc)
        out = jnp.dot(a, wp_ref[:, :], preferred_element_type=jnp.float32)
        out_ref[pl.ds(r0, h), :] = out.astype(CDT)

    loads = [start_load(0, 0), None]
    pl.semaphore_wait(barrier_sem, 3)

    rdmas0 = []
    for b in range(6):
        slot = b % 2
        if b < 5:
            loads[1 - slot] = start_load(b + 1, 1 - slot)
        loads[slot].wait()
        compute_block(b, slot)
        if b < 3:
            p = b
            mask = MASKS[p][0]
            rdma = pltpu.make_async_remote_copy(
                src_ref=out_ref.at[pl.ds(send_starts[p], lens[p]), :],
                dst_ref=buf_ref.at[p, pl.ds(0, lens[p]), :],
                send_sem=rs_send.at[p, 0],
                recv_sem=rs_recv.at[p, 0],
                device_id=(jnp.bitwise_xor(my, mask),),
                device_id_type=pl.DeviceIdType.MESH,
            )
            rdma.start()
            rdmas0.append(rdma)
    for p in range(3):
        rdmas0[p].wait()
        h = lens[p]
        rows = pl.ds(los[p], h)
        out_ref[rows, :] = out_ref[rows, :] + buf_ref[p, pl.ds(0, h), :]

    for s in (1, 2):
        rdmas = []
        for p in range(3):
            mask = MASKS[p][s]
            q = jnp.bitwise_xor(my, mask)
            side = side_of[mask] == 1
            h = lens[p] // 2
            send_start = los[p] + jnp.where(side, 0, h)
            rdma = pltpu.make_async_remote_copy(
                src_ref=out_ref.at[pl.ds(send_start, h), :],
                dst_ref=buf_ref.at[p, pl.ds(0, h), :],
                send_sem=rs_send.at[p, s],
                recv_sem=rs_recv.at[p, s],
                device_id=(q,),
                device_id_type=pl.DeviceIdType.MESH,
            )
            rdma.start()
            rdmas.append(rdma)
            los[p] = los[p] + jnp.where(side, h, 0)
            lens[p] = h
        for p in range(3):
            rdmas[p].wait()
            h = lens[p]
            rows = pl.ds(los[p], h)
            out_ref[rows, :] = out_ref[rows, :] + buf_ref[p, pl.ds(0, h), :]

    for s in (2, 1, 0):
        rdmas = []
        for p in range(3):
            mask = MASKS[p][s]
            q = jnp.bitwise_xor(my, mask)
            side = side_of[mask] == 1
            L = lens[p]
            rdma = pltpu.make_async_remote_copy(
                src_ref=out_ref.at[pl.ds(los[p], L), :],
                dst_ref=out_ref.at[pl.ds(los[p], L), :],
                send_sem=ag_send.at[p, s],
                recv_sem=ag_recv.at[p, s],
                device_id=(q,),
                device_id_type=pl.DeviceIdType.MESH,
            )
            rdma.start()
            rdmas.append(rdma)
            los[p] = los[p] - jnp.where(side, L, 0)
            lens[p] = 2 * L
        for p in range(3):
            rdmas[p].wait()


def kernel(x, k, Wp):
    b, seq, c = x.shape
    reduced = pl.pallas_call(
        _body,
        out_shape=jax.ShapeDtypeStruct((ROWS, COLS), CDT),
        in_specs=[
            pl.BlockSpec(memory_space=pl.ANY),
            pl.BlockSpec(memory_space=pltpu.VMEM),
            pl.BlockSpec(memory_space=pltpu.VMEM),
        ],
        out_specs=pl.BlockSpec(memory_space=pltpu.VMEM),
        scratch_shapes=[
            pltpu.VMEM((2, STAG_ROWS, COLS), jnp.float32),
            pltpu.VMEM((3, BUF_ROWS, COLS), CDT),
            pltpu.SemaphoreType.DMA((2,)),
            pltpu.SemaphoreType.DMA((3, 3)),
            pltpu.SemaphoreType.DMA((3, 3)),
            pltpu.SemaphoreType.DMA((3, 3)),
            pltpu.SemaphoreType.DMA((3, 3)),
        ],
        compiler_params=pltpu.CompilerParams(
            collective_id=0, vmem_limit_bytes=62 * 1024 * 1024,
        ),
    )(x.reshape(b * seq, c), k.astype(CDT), Wp.astype(CDT))
    return reduced.reshape(b, seq, COLS)
